# baseline (speedup 1.0000x reference)
import functools

import jax
import jax.numpy as jnp
from jax import lax
from jax.experimental import pallas as pl
from jax.experimental.pallas import tpu as pltpu
from jax.experimental.pallas import tpu_sc as plsc

_S = 5.0
_LN2 = 0.6931471805599453
_N = 4096
_NW = 32
_RPW = _N // _NW
_NG = _RPW // 16
_NV = _N // 16

_C5 = 4.39286278e-02
_C4 = -1.89832447e-01
_C3 = 4.11561482e-01
_C2 = -7.07253434e-01
_C1 = 1.44159208e+00
_C0 = 1.43909300e-05


def _sc_body(z_hbm, y_hbm, outs_hbm, outc_hbm, sz_v, y_v, res_s, res_c):
    wid = lax.axis_index("s") * 2 + lax.axis_index("c")
    base = wid * _RPW
    pltpu.sync_copy(z_hbm, sz_v.at[pl.ds(0, _N)])
    pltpu.sync_copy(y_hbm, y_v.at[pl.ds(0, _N)])

    def scale_body(v, carry):
        sl = pl.ds(v * 16, 16)
        sz_v[sl] = sz_v[sl] * jnp.float32(_S)
        return carry

    lax.fori_loop(0, _NV, scale_body, 0)
    sz_v[pl.ds(_N, 16)] = sz_v[pl.ds(0, 16)]
    y_v[pl.ds(_N, 16)] = y_v[pl.ds(0, 16)]

    def group_body(rg, accs):
        row0 = base + rg * 16
        zrow = sz_v[pl.ds(row0, 16)]
        yrow = y_v[pl.ds(row0, 16)]

        def col_body(k, accs2):
            s2, c2 = accs2
            zj = sz_v[pl.ds(k, 16)]
            yj = y_v[pl.ds(k, 16)]
            dy = yrow - yj
            dz = zj - zrow
            sb = lax.bitcast_convert_type(dy, jnp.uint32) & jnp.uint32(0x80000000)
            a = lax.bitcast_convert_type(
                lax.bitcast_convert_type(dz, jnp.uint32) ^ sb, jnp.float32)
            e = jnp.exp(a)
            w = 1.0 + e
            wb = lax.bitcast_convert_type(w, jnp.uint32)
            ex = (lax.shift_right_logical(wb, jnp.uint32(23)).astype(jnp.int32)
                  - 127).astype(jnp.float32)
            t = lax.bitcast_convert_type(
                (wb & jnp.uint32(0x007FFFFF)) | jnp.uint32(0x3F800000),
                jnp.float32) - 1.0
            p = ((((jnp.float32(_C5) * t + jnp.float32(_C4)) * t
                   + jnp.float32(_C3)) * t + jnp.float32(_C2)) * t
                 + jnp.float32(_C1)) * t + jnp.float32(_C0)
            v2 = ex + p
            mask = dy != 0.0
            s2 = s2 + jnp.where(mask, v2, 0.0)
            c2 = c2 + jnp.where(mask, 1.0, 0.0)
            return (s2, c2)

        return lax.fori_loop(0, _N, col_body, accs2 := accs)

    zero = jnp.zeros((16,), jnp.float32)
    s_acc, c_acc = lax.fori_loop(0, _NG, group_body, (zero, zero))
    res_s[...] = s_acc * jnp.float32(_LN2)
    res_c[...] = c_acc
    pltpu.sync_copy(res_s, outs_hbm.at[wid])
    pltpu.sync_copy(res_c, outc_hbm.at[wid])


@jax.jit
def kernel(z, y):
    z = z.reshape(-1)
    y = y.reshape(-1)
    mesh = plsc.VectorSubcoreMesh(core_axis_name="c", subcore_axis_name="s")
    sck = functools.partial(
        pl.kernel,
        mesh=mesh,
        out_type=[
            jax.ShapeDtypeStruct((_NW, 16), jnp.float32),
            jax.ShapeDtypeStruct((_NW, 16), jnp.float32),
        ],
        scratch_types=[
            pltpu.VMEM((_N + 16,), jnp.float32),
            pltpu.VMEM((_N + 16,), jnp.float32),
            pltpu.VMEM((16,), jnp.float32),
            pltpu.VMEM((16,), jnp.float32),
        ],
    )(_sc_body)
    outs, outc = sck(z, y)
    s = jnp.sum(outs)
    c = jnp.sum(outc)
    return jnp.where(c > 0, s / jnp.maximum(c, 1.0), 0.0)


# speedup vs baseline: 18.1246x; 18.1246x over previous
import jax
import jax.numpy as jnp
from jax import lax
from jax.experimental import pallas as pl

_S = 5.0
_LOG2E = 1.4426950408889634
_N = 4096
_B = 128
_NB = _N // _B
_SIGNBIT = 0x80000000


def _body(zr_ref, yr_ref, loss_ref):
    alpha = jnp.float32(_S * _LOG2E)
    sz = zr_ref[...] * alpha
    yy = yr_ref[...]
    ones = jnp.ones((1, _B), jnp.float32)
    acc_s = jnp.zeros((1, _B), jnp.float32)
    acc_c = jnp.zeros((1, _B), jnp.float32)
    for bi in range(_NB):
        szi = lax.transpose(sz[:, bi * _B:(bi + 1) * _B], (1, 0))
        yi = lax.transpose(yy[:, bi * _B:(bi + 1) * _B], (1, 0))
        for bj in range(bi, _NB):
            szj = sz[:, bj * _B:(bj + 1) * _B]
            yj = yy[:, bj * _B:(bj + 1) * _B]
            dy = yi - yj
            dz = szj - szi
            sbit = lax.bitcast_convert_type(dy, jnp.uint32) & jnp.uint32(_SIGNBIT)
            a = lax.bitcast_convert_type(
                lax.bitcast_convert_type(dz, jnp.uint32) ^ sbit, jnp.float32)
            vals = jnp.log(1.0 + jnp.exp2(a))
            mask = dy != 0.0
            vals_m = jnp.where(mask, vals, 0.0)
            mask_f = jnp.where(mask, 1.0, 0.0)
            rs = jnp.dot(ones, vals_m, preferred_element_type=jnp.float32)
            rc = jnp.dot(ones, mask_f, preferred_element_type=jnp.float32)
            if bi == bj:
                rs = rs * 0.5
                rc = rc * 0.5
            acc_s = acc_s + rs
            acc_c = acc_c + rc
    s = jnp.sum(acc_s, keepdims=True)
    c = jnp.sum(acc_c, keepdims=True)
    loss_ref[...] = jnp.where(c > 0, s / jnp.maximum(c, 1.0), 0.0)


@jax.jit
def kernel(z, y):
    z = z.reshape(-1)
    y = y.reshape(-1)
    loss = pl.pallas_call(
        _body,
        out_shape=jax.ShapeDtypeStruct((1, 1), jnp.float32),
    )(
        z.reshape(1, _N),
        y.reshape(1, _N),
    )
    return loss[0, 0]
